# Initial kernel scaffold; baseline (speedup 1.0000x reference)
import jax
import jax.numpy as jnp
from jax.experimental import pallas as pl


def kernel(x, edge_index, node_type, batch, goal_embedding, params):
    raise NotImplementedError("write your pallas kernel here")


# speedup vs baseline: 31.6601x; 31.6601x over previous
import functools

import jax
import jax.numpy as jnp
from jax import lax
from jax.experimental import pallas as pl
from jax.experimental.pallas import tpu as pltpu
from jax.experimental.pallas import tpu_sc as plsc

N = 50000
E = 800000
ETOT = E + N
B = 8
H = 64
HEADS = 4
OC = 16
GD = 64

NS = 16
NC = 2
CH = 80
NJ = -(-ETOT // (CH * NS))
EP = NJ * NS * CH
HALF = N // NC
ROWS = HALF + 24
RPT = ROWS // NS
AW = 72

BLK = 2000
NBLK = N // BLK


def _scatter_add(obuf, acc, locv):
    pltpu.sync_copy(obuf, acc.at[locv], add=True)


def _make_sc_edge():
    mesh = plsc.VectorSubcoreMesh(core_axis_name="c", subcore_axis_name="s",
                                  num_cores=NC, num_subcores=NS)

    @functools.partial(
        pl.kernel,
        out_type=jax.ShapeDtypeStruct((NC, ROWS, AW), jnp.float32),
        mesh=mesh,
        scratch_types=[
            pltpu.VMEM((CH,), jnp.int32),
            pltpu.VMEM((CH,), jnp.int32),
            pltpu.VMEM((CH,), jnp.int32),
            pltpu.VMEM((CH,), jnp.float32),
            pltpu.VMEM((HEADS, OC), jnp.float32),
            pltpu.VMEM((CH, H), jnp.float32),
            pltpu.VMEM((CH, H), jnp.float32),
            pltpu.VMEM((CH, AW), jnp.float32),
            pltpu.VMEM_SHARED((ROWS, AW), jnp.float32),
            pltpu.SemaphoreType.DMA,
            pltpu.SemaphoreType.DMA,
        ],
        compiler_params=pltpu.CompilerParams(needs_layout_passes=False,
                                             use_tc_tiling_on_sc=False),
    )
    def sc_edge(xl, xr, srcp, dstp, att, out,
                srcv, dstv, locv, vb, attv, xlb, xrb, obuf, acc, sem1, sem2):
        c = lax.axis_index("c")
        s = lax.axis_index("s")
        base = c * HALF
        lane = lax.iota(jnp.int32, 16)
        zv = jnp.zeros((16,), jnp.float32)

        pltpu.sync_copy(att, attv)
        atth = [attv[h] for h in range(HEADS)]

        def zrow(r, carry):
            for cc in (0, 16, 32, 48, 56):
                obuf[r, pl.ds(cc, 16)] = zv
            return carry
        lax.fori_loop(0, CH, zrow, 0)
        nfull = RPT // CH
        rem = RPT - nfull * CH
        for t in range(nfull):
            pltpu.sync_copy(obuf, acc.at[pl.ds(s * RPT + t * CH, CH)])
        if rem:
            pltpu.sync_copy(obuf.at[pl.ds(0, rem)],
                            acc.at[pl.ds(s * RPT + nfull * CH, rem)])
        plsc.subcore_barrier()

        def chunk(j, carry):
            eb = (j * NS + s) * CH
            cp1 = pltpu.async_copy(srcp.at[pl.ds(eb, CH)], srcv, sem1)
            cp2 = pltpu.async_copy(dstp.at[pl.ds(eb, CH)], dstv, sem2)
            cp1.wait()
            cp2.wait()
            cp1 = pltpu.async_copy(xl.at[srcv], xlb, sem1)
            cp2 = pltpu.async_copy(xr.at[dstv], xrb, sem2)
            for g in range(CH // 16):
                dv = dstv[pl.ds(g * 16, 16)]
                ld = dv - base
                ev = lane + (eb + g * 16)
                valid = (ev < ETOT) & (ld >= 0) & (ld < HALF)
                locv[pl.ds(g * 16, 16)] = jnp.where(valid, ld, ld & 8191)
                vb[pl.ds(g * 16, 16)] = jnp.where(valid, 0.0, -1e30)
            cp1.wait()
            cp2.wait()

            def grp(g, gcarry):
                bv = vb[pl.ds(g * 16, 16)]
                for l in range(16):
                    e = g * 16 + l
                    bias = bv[l]
                    exs = []
                    xls = []
                    for h in range(HEADS):
                        xlh = xlb[e, pl.ds(h * 16, 16)]
                        xrh = xrb[e, pl.ds(h * 16, 16)]
                        sh = xlh + xrh
                        th = jnp.maximum(sh, 0.2 * sh)
                        cs = plsc.cumsum(th * atth[h])
                        ex = jnp.exp(lax.broadcast(cs[15] + bias, (16,)))
                        exs.append(ex)
                        xls.append(xlh)
                    ev = jnp.where(lane >= 1, exs[1], exs[0])
                    ev = jnp.where(lane >= 2, exs[2], ev)
                    ev = jnp.where(lane >= 3, exs[3], ev)
                    obuf[e, pl.ds(0, 16)] = ev
                    for h in range(HEADS):
                        obuf[e, pl.ds(8 + h * 16, 16)] = exs[h] * xls[h]
                return gcarry
            lax.fori_loop(0, CH // 16, grp, 0)
            _scatter_add(obuf, acc, locv)
            return carry
        lax.fori_loop(0, NJ, chunk, 0)

        plsc.subcore_barrier()
        pltpu.sync_copy(acc.at[pl.ds(s * RPT, RPT)],
                        out.at[c, pl.ds(s * RPT, RPT)])

    return sc_edge


_sc_edge = _make_sc_edge()


def _ln(v, g, b):
    mu = jnp.mean(v, axis=-1, keepdims=True)
    var = jnp.mean((v - mu) ** 2, axis=-1, keepdims=True)
    return (v - mu) * lax.rsqrt(var + 1e-5) * g + b


def _enc_body(xb, ntb, w1c, b1c, g1c, q1c, w2c, b2c, g2c, q2c, w3c, b3c,
              w1g, b1g, g1g, q1g, w2g, b2g, temb, ho):
    x_ = xb[...]
    cv = jax.nn.relu(_ln(jnp.dot(x_, w1c[...]) + b1c[...], g1c[...], q1c[...]))
    cv = jax.nn.relu(_ln(jnp.dot(cv, w2c[...]) + b2c[...], g2c[...], q2c[...]))
    cv = jnp.dot(cv, w3c[...]) + b3c[...]
    gv = jax.nn.relu(_ln(jnp.dot(x_, w1g[...]) + b1g[...], g1g[...], q1g[...]))
    gv = jnp.dot(gv, w2g[...]) + b2g[...]
    nt = ntb[...]
    io8 = lax.broadcasted_iota(jnp.int32, (1, 8), 1)
    oh = (nt == io8).astype(jnp.float32)
    emb = jnp.dot(oh, temb[...])
    hsel = jnp.where(nt == 0, cv, jnp.where(nt == 1, gv, 0.0))
    ho[...] = hsel + emb


def _proj_body(hb, w, xo):
    xo[...] = jnp.dot(hb[...], w[...])


def _post_body(hb, aggb, btb, bias, lg, lb, goal, fgw, fgb, fbw, fbb, e8, ho):
    agg = aggb[...]
    num = agg[:, 8:72]
    den4 = agg[:, 0:4]
    dm = jnp.dot(den4, e8[...][0:4, :])
    msg = num / (dm + 1e-16) + bias[...]
    t = _ln(hb[...] + msg, lg[...], lb[...])
    bt_ = btb[...]
    io8 = lax.broadcasted_iota(jnp.int32, (1, 8), 1)
    oh = (bt_ == io8).astype(jnp.float32)
    gm = jnp.dot(oh, jnp.dot(goal[...], fgw[...]) + fgb[...] + 1.0)
    be = jnp.dot(oh, jnp.dot(goal[...], fbw[...]) + fbb[...])
    ho[...] = jax.nn.relu(t * gm + be)


def _pool_body(hb, btb, w1, b1, w2, b2, out, ssum, smax):
    i = pl.program_id(0)
    h_ = hb[...]
    bt_ = btb[...]
    io8 = lax.broadcasted_iota(jnp.int32, (1, 8), 1)
    oh = (bt_ == io8).astype(jnp.float32)
    hext = jnp.concatenate([h_, jnp.ones((BLK, 16), jnp.float32)], axis=1)
    part = lax.dot_general(oh, hext, (((0,), (0,)), ((), ())),
                           preferred_element_type=jnp.float32)

    @pl.when(i == 0)
    def _():
        ssum[...] = jnp.zeros((B, 80), jnp.float32)
        smax[...] = jnp.full((B, H), -jnp.inf, jnp.float32)

    ssum[...] += part
    for b in range(B):
        mask = bt_ == b
        hm = jnp.where(mask, h_, -jnp.inf)
        mb = jnp.max(hm, axis=0, keepdims=True)
        smax[b:b + 1, :] = jnp.maximum(smax[b:b + 1, :], mb)

    @pl.when(i == NBLK - 1)
    def _():
        cnt = ssum[:, 64:65]
        mean = ssum[:, 0:64] / jnp.maximum(cnt, 1.0)
        gfin = jnp.concatenate([mean, smax[...]], axis=1)
        z = jax.nn.relu(jnp.dot(gfin, w1[...]) + b1[...])
        out[...] = jnp.dot(z, w2[...]) + b2[...]


def _full(shape):
    return pl.BlockSpec(shape, lambda i: tuple(0 for _ in shape))


def _enc_call(xp, ntb, *ws):
    specs = [pl.BlockSpec((BLK, 32), lambda i: (i, 0)),
             pl.BlockSpec((BLK, 1), lambda i: (i, 0))]
    specs += [_full(w.shape) for w in ws]
    return pl.pallas_call(
        _enc_body, grid=(NBLK,), in_specs=specs,
        out_specs=pl.BlockSpec((BLK, H), lambda i: (i, 0)),
        out_shape=jax.ShapeDtypeStruct((N, H), jnp.float32),
    )(xp, ntb, *ws)


def _proj_call(h, w):
    return pl.pallas_call(
        _proj_body, grid=(NBLK,),
        in_specs=[pl.BlockSpec((BLK, H), lambda i: (i, 0)), _full(w.shape)],
        out_specs=pl.BlockSpec((BLK, H), lambda i: (i, 0)),
        out_shape=jax.ShapeDtypeStruct((N, H), jnp.float32),
    )(h, w)


def _post_call(h, agg, btb, *ws):
    specs = [pl.BlockSpec((BLK, H), lambda i: (i, 0)),
             pl.BlockSpec((BLK, AW), lambda i: (i, 0)),
             pl.BlockSpec((BLK, 1), lambda i: (i, 0))]
    specs += [_full(w.shape) for w in ws]
    return pl.pallas_call(
        _post_body, grid=(NBLK,), in_specs=specs,
        out_specs=pl.BlockSpec((BLK, H), lambda i: (i, 0)),
        out_shape=jax.ShapeDtypeStruct((N, H), jnp.float32),
    )(h, agg, btb, *ws)


def _pool_call(h, btb, w1, b1, w2, b2):
    return pl.pallas_call(
        _pool_body, grid=(NBLK,),
        in_specs=[pl.BlockSpec((BLK, H), lambda i: (i, 0)),
                  pl.BlockSpec((BLK, 1), lambda i: (i, 0)),
                  _full(w1.shape), _full(b1.shape),
                  _full(w2.shape), _full(b2.shape)],
        out_specs=pl.BlockSpec((B, H), lambda i: (0, 0)),
        out_shape=jax.ShapeDtypeStruct((B, H), jnp.float32),
        scratch_shapes=[pltpu.VMEM((B, 80), jnp.float32),
                        pltpu.VMEM((B, H), jnp.float32)],
    )(h, btb, w1, b1, w2, b2)


def _row(v):
    return v.reshape(1, -1)


@jax.jit
def _run(x, edge_index, node_type, batch, goal_embedding, params):
    f32 = jnp.float32
    xp = jnp.pad(x.astype(f32), ((0, 0), (0, 7)))
    ntb = node_type.astype(jnp.int32).reshape(N, 1)
    btb = batch.astype(jnp.int32).reshape(N, 1)

    loop = jnp.arange(N, dtype=jnp.int32)
    zpad = jnp.zeros((EP - ETOT,), jnp.int32)
    srcp = jnp.concatenate([edge_index[0].astype(jnp.int32), loop, zpad])
    dstp = jnp.concatenate([edge_index[1].astype(jnp.int32), loop, zpad])

    pc, pg = params["cube"], params["group"]
    w1c = jnp.pad(pc["l1"]["W"], ((0, 7), (0, 0)))
    w1g = jnp.pad(pg["l1"]["W"], ((0, 20), (0, 0)))
    temb = jnp.pad(params["type_emb"], ((0, 5), (0, 0)))
    h = _enc_call(
        xp, ntb,
        w1c, _row(pc["l1"]["b"]), _row(pc["ln1"]["g"]), _row(pc["ln1"]["b"]),
        pc["l2"]["W"], _row(pc["l2"]["b"]), _row(pc["ln2"]["g"]), _row(pc["ln2"]["b"]),
        pc["l3"]["W"], _row(pc["l3"]["b"]),
        w1g, _row(pg["l1"]["b"]), _row(pg["ln1"]["g"]), _row(pg["ln1"]["b"]),
        pg["l2"]["W"], _row(pg["l2"]["b"]),
        temb)

    co = jnp.arange(64) // 16
    e8 = (co[None, :] == jnp.arange(8)[:, None]).astype(f32)

    goal = goal_embedding.astype(f32)
    for lp in params["layers"]:
        xl = _proj_call(h, lp["Wl"])
        xr = _proj_call(h, lp["Wr"])
        out_sc = _sc_edge(xl, xr, srcp, dstp, lp["att"])
        agg = out_sc[:, :HALF, :].reshape(N, AW)
        h = _post_call(
            h, agg, btb,
            _row(lp["bias"]), _row(lp["ln"]["g"]), _row(lp["ln"]["b"]),
            goal, lp["film_gW"], _row(lp["film_gb"]),
            lp["film_bW"], _row(lp["film_bb"]), e8)

    gf = _pool_call(h, btb, params["out1"]["W"], _row(params["out1"]["b"]),
                    params["out2"]["W"], _row(params["out2"]["b"]))
    return gf, h


def kernel(x, edge_index, node_type, batch, goal_embedding, params):
    return _run(x, edge_index, node_type, batch, goal_embedding, params)
